# baseline (speedup 1.0000x reference)
import functools

import jax
import jax.numpy as jnp
from jax import lax
from jax.experimental import pallas as pl
from jax.experimental.pallas import tpu as pltpu
from jax.experimental.pallas import tpu_sc as plsc

N = 10000
E = 320000
DF = 128
DE = 16
H = 128
EMB = 128
G = 64

NC = 2
NS = 16
CHUNK = 64
NCHUNKS = E // CHUNK
CPW = -(-NCHUNKS // (NC * NS))
ROWS_PER_SUB = 624
TAIL_ROWS = N - NS * ROWS_PER_SUB


NW = NC * NS
NBI = 3
NBH = 2
UNROLL = 6
LOOP_HI = UNROLL * (-(-(CPW + 2) // UNROLL))


def _edge_pass_body(h_hbm, e_hbm, src_hbm, dst_hbm, out_hbm,
                    acc, src_v, dst_v, h_v, e_v, sem_in, sem_g):
    c = lax.axis_index("core")
    s = lax.axis_index("subcore")
    w = c * NS + s

    @pl.loop(0, CHUNK)
    def _(r):
        for k in range(H // 16):
            e_v[0, pl.ds(r, 1), pl.ds(k * 16, 16)] = jnp.zeros((1, 16), jnp.float32)

    _nfull, _rem = divmod(ROWS_PER_SUB, CHUNK)
    for j in range(_nfull):
        pltpu.sync_copy(e_v.at[0],
                        acc.at[pl.ds(s * ROWS_PER_SUB + j * CHUNK, CHUNK)])
    if _rem:
        pltpu.sync_copy(e_v.at[0, pl.ds(0, _rem)],
                        acc.at[pl.ds(s * ROWS_PER_SUB + _nfull * CHUNK, _rem)])

    @pl.when(s == 0)
    def _():
        pltpu.sync_copy(e_v.at[0, pl.ds(0, TAIL_ROWS)],
                        acc.at[pl.ds(NS * ROWS_PER_SUB, TAIL_ROWS)])

    plsc.subcore_barrier()

    def start_in(i, ji):
        @pl.when(w + NW * i < NCHUNKS)
        def _():
            base = (w + NW * i) * CHUNK
            pltpu.async_copy(src_hbm.at[pl.ds(base, CHUNK)], src_v.at[ji],
                             sem_in.at[ji])
            pltpu.async_copy(dst_hbm.at[pl.ds(base, CHUNK)], dst_v.at[ji],
                             sem_in.at[ji])
            pltpu.async_copy(e_hbm.at[pl.ds(base, CHUNK)], e_v.at[ji],
                             sem_in.at[ji])

    def start_gather(i, ji, jh):
        @pl.when(w + NW * i < NCHUNKS)
        def _():
            pltpu.make_async_copy(src_hbm.at[pl.ds(0, CHUNK)], src_v.at[ji],
                                  sem_in.at[ji]).wait()
            pltpu.make_async_copy(dst_hbm.at[pl.ds(0, CHUNK)], dst_v.at[ji],
                                  sem_in.at[ji]).wait()
            pltpu.make_async_copy(e_hbm.at[pl.ds(0, CHUNK)], e_v.at[ji],
                                  sem_in.at[ji]).wait()
            pltpu.async_copy(h_hbm.at[src_v.at[ji]], h_v.at[jh], sem_g.at[jh])

    start_in(0, 0)
    start_in(1, 1)
    start_gather(0, 0, 0)

    @pl.loop(0, LOOP_HI, step=UNROLL)
    def _(i0):
        for k in range(UNROLL):
            i = i0 + k
            ji, jh = k % NBI, k % NBH
            start_in(i + 2, (k + 2) % NBI)
            start_gather(i + 1, (k + 1) % NBI, (k + 1) % NBH)

            @pl.when(w + NW * i < NCHUNKS)
            def _():
                pltpu.make_async_copy(h_hbm.at[src_v.at[ji]], h_v.at[jh],
                                      sem_g.at[jh]).wait()

                @pl.loop(0, CHUNK, step=2)
                def _(r):
                    for rr in range(2):
                        for kk in range(H // 16):
                            sl = (jh, pl.ds(r + rr, 1), pl.ds(kk * 16, 16))
                            h_v[sl] = jnp.maximum(h_v[sl] + e_v[ji, sl[1], sl[2]],
                                                  0.0)

                pltpu.sync_copy(h_v.at[jh], acc.at[dst_v.at[ji]], add=True)

    plsc.subcore_barrier()

    pltpu.sync_copy(acc.at[pl.ds(s * ROWS_PER_SUB, ROWS_PER_SUB)],
                    out_hbm.at[c, pl.ds(s * ROWS_PER_SUB, ROWS_PER_SUB)])

    @pl.when(s == 0)
    def _():
        pltpu.sync_copy(acc.at[pl.ds(NS * ROWS_PER_SUB, TAIL_ROWS)],
                        out_hbm.at[c, pl.ds(NS * ROWS_PER_SUB, TAIL_ROWS)])


@jax.jit
def _edge_pass(h, e, src, dst):
    mesh = plsc.VectorSubcoreMesh(core_axis_name="core", subcore_axis_name="subcore")
    f = pl.kernel(
        _edge_pass_body,
        out_type=jax.ShapeDtypeStruct((NC, N, H), jnp.float32),
        mesh=mesh,
        scratch_types=[
            pltpu.VMEM_SHARED((N, H), jnp.float32),
            pltpu.VMEM((NBI, CHUNK), jnp.int32),
            pltpu.VMEM((NBI, CHUNK), jnp.int32),
            pltpu.VMEM((NBH, CHUNK, H), jnp.float32),
            pltpu.VMEM((NBI, CHUNK, H), jnp.float32),
            pltpu.SemaphoreType.DMA((NBI,)),
            pltpu.SemaphoreType.DMA((NBH,)),
        ],
        name="gine_edge_pass",
    )
    return f(h, e, src, dst)


def _encode_h_body(x_ref, wn_ref, bn_ref, out_ref):
    out_ref[...] = jnp.dot(x_ref[...], wn_ref[...],
                           preferred_element_type=jnp.float32,
                           precision=lax.Precision.HIGHEST) + bn_ref[...]


def _encode_e_body(ea_ref, we_ref, be_ref, out_ref):
    out_ref[...] = jnp.dot(ea_ref[...], we_ref[...],
                           preferred_element_type=jnp.float32,
                           precision=lax.Precision.HIGHEST) + be_ref[...]


def _mlp_core(h_ref, parts_ref, w1_ref, b1_ref, w2_ref, b2_ref, g_ref, bt_ref):
    z = h_ref[...] + parts_ref[0] + parts_ref[1]
    z = jnp.maximum(jnp.dot(z, w1_ref[...], preferred_element_type=jnp.float32,
                            precision=lax.Precision.HIGHEST) + b1_ref[...], 0.0)
    z = jnp.dot(z, w2_ref[...], preferred_element_type=jnp.float32,
                precision=lax.Precision.HIGHEST) + b2_ref[...]
    mu = jnp.mean(z, axis=0, keepdims=True)
    var = jnp.mean(z * z, axis=0, keepdims=True) - mu * mu
    zn = (z - mu) * lax.rsqrt(var + 1e-5) * g_ref[...] + bt_ref[...]
    return jnp.maximum(zn, 0.0)


def _mlp_bn_body(h_ref, parts_ref, w1_ref, b1_ref, w2_ref, b2_ref,
                 g_ref, bt_ref, out_ref):
    out_ref[...] = _mlp_core(h_ref, parts_ref, w1_ref, b1_ref, w2_ref, b2_ref,
                             g_ref, bt_ref)


def _mlp_bn_pool_body(h_ref, parts_ref, w1_ref, b1_ref, w2_ref, b2_ref,
                      g_ref, bt_ref, b_ref, wf_ref, bf_ref, out_ref):
    hf = _mlp_core(h_ref, parts_ref, w1_ref, b1_ref, w2_ref, b2_ref,
                   g_ref, bt_ref)
    gids = lax.broadcasted_iota(jnp.int32, (G, N), 0)
    oh = (gids == b_ref[...]).astype(jnp.float32)
    sums = jnp.dot(oh, hf, preferred_element_type=jnp.float32,
                   precision=lax.Precision.HIGHEST)
    cnt = jnp.sum(oh, axis=1, keepdims=True)
    mean = sums / jnp.maximum(cnt, 1.0)
    out_ref[...] = jnp.dot(mean, wf_ref[...], preferred_element_type=jnp.float32,
                           precision=lax.Precision.HIGHEST) + bf_ref[...]


def _tc_call(body, out_shape, *args):
    return pl.pallas_call(body, out_shape=out_shape)(*args)


def kernel(x, edge_index, edge_attr, batch, Wn, bn_, We, be,
           l0_w1, l0_b1, l0_w2, l0_b2, l0_g, l0_bt,
           l1_w1, l1_b1, l1_w2, l1_b2, l1_g, l1_bt, Wf, bf):
    src = edge_index[0]
    dst = edge_index[1]

    h = _tc_call(_encode_h_body, jax.ShapeDtypeStruct((N, H), jnp.float32),
                 x, Wn, bn_.reshape(1, H))

    PK = 8
    W8 = jnp.zeros((PK, DE, PK, H), jnp.float32)
    W8 = W8.at[jnp.arange(PK), :, jnp.arange(PK), :].set(
        jnp.broadcast_to(We, (PK, DE, H))).reshape(PK * DE, PK * H)
    EB = 2000
    e = pl.pallas_call(
        _encode_e_body,
        out_shape=jax.ShapeDtypeStruct((E // PK, PK * H), jnp.float32),
        grid=(E // PK // EB,),
        in_specs=[pl.BlockSpec((EB, PK * DE), lambda i: (i, 0)),
                  pl.BlockSpec((PK * DE, PK * H), lambda i: (0, 0)),
                  pl.BlockSpec((1, PK * H), lambda i: (0, 0))],
        out_specs=pl.BlockSpec((EB, PK * H), lambda i: (i, 0)),
    )(edge_attr.reshape(E // PK, PK * DE), W8,
      jnp.tile(be, (PK,)).reshape(1, PK * H)).reshape(E, H)

    (w1, b1, w2, b2, g, bt) = (l0_w1, l0_b1, l0_w2, l0_b2, l0_g, l0_bt)
    parts = _edge_pass(h, e, src, dst)
    h = _tc_call(_mlp_bn_body, jax.ShapeDtypeStruct((N, H), jnp.float32),
                 h, parts, w1, b1.reshape(1, H), w2, b2.reshape(1, H),
                 g.reshape(1, H), bt.reshape(1, H))

    (w1, b1, w2, b2, g, bt) = (l1_w1, l1_b1, l1_w2, l1_b2, l1_g, l1_bt)
    parts = _edge_pass(h, e, src, dst)
    return _tc_call(_mlp_bn_pool_body, jax.ShapeDtypeStruct((G, EMB), jnp.float32),
                    h, parts, w1, b1.reshape(1, H), w2, b2.reshape(1, H),
                    g.reshape(1, H), bt.reshape(1, H),
                    batch.reshape(1, N), Wf, bf.reshape(1, EMB))


# speedup vs baseline: 1.0996x; 1.0996x over previous
import functools

import jax
import jax.numpy as jnp
from jax import lax
from jax.experimental import pallas as pl
from jax.experimental.pallas import tpu as pltpu
from jax.experimental.pallas import tpu_sc as plsc

N = 10000
E = 320000
DF = 128
DE = 16
H = 128
EMB = 128
G = 64

NC = 2
NS = 16
CHUNK = 64
NCHUNKS = E // CHUNK
CPW = -(-NCHUNKS // (NC * NS))
ROWS_PER_SUB = 624
TAIL_ROWS = N - NS * ROWS_PER_SUB


NW = NC * NS
NBI = 3
NBH = 2
UNROLL = 6
LOOP_HI = UNROLL * (-(-(CPW + 2) // UNROLL))


def _edge_pass_body(h_hbm, e_hbm, src_hbm, dst_hbm, out_hbm,
                    acc, src_v, dst_v, h_v, e_v, sem_in, sem_g):
    c = lax.axis_index("core")
    s = lax.axis_index("subcore")
    w = c * NS + s

    @pl.loop(0, CHUNK)
    def _(r):
        for k in range(H // 16):
            e_v[0, pl.ds(r, 1), pl.ds(k * 16, 16)] = jnp.zeros((1, 16), jnp.float32)

    _nfull, _rem = divmod(ROWS_PER_SUB, CHUNK)
    for j in range(_nfull):
        pltpu.sync_copy(e_v.at[0],
                        acc.at[pl.ds(s * ROWS_PER_SUB + j * CHUNK, CHUNK)])
    if _rem:
        pltpu.sync_copy(e_v.at[0, pl.ds(0, _rem)],
                        acc.at[pl.ds(s * ROWS_PER_SUB + _nfull * CHUNK, _rem)])

    @pl.when(s == 0)
    def _():
        pltpu.sync_copy(e_v.at[0, pl.ds(0, TAIL_ROWS)],
                        acc.at[pl.ds(NS * ROWS_PER_SUB, TAIL_ROWS)])

    plsc.subcore_barrier()

    def start_in(i, ji):
        @pl.when(w + NW * i < NCHUNKS)
        def _():
            base = (w + NW * i) * CHUNK
            pltpu.async_copy(src_hbm.at[pl.ds(base, CHUNK)], src_v.at[ji],
                             sem_in.at[ji])
            pltpu.async_copy(dst_hbm.at[pl.ds(base, CHUNK)], dst_v.at[ji],
                             sem_in.at[ji])
            pltpu.async_copy(e_hbm.at[pl.ds(base, CHUNK)], e_v.at[ji],
                             sem_in.at[ji])

    def start_gather(i, ji, jh):
        @pl.when(w + NW * i < NCHUNKS)
        def _():
            pltpu.make_async_copy(src_hbm.at[pl.ds(0, CHUNK)], src_v.at[ji],
                                  sem_in.at[ji]).wait()
            pltpu.make_async_copy(dst_hbm.at[pl.ds(0, CHUNK)], dst_v.at[ji],
                                  sem_in.at[ji]).wait()
            pltpu.make_async_copy(e_hbm.at[pl.ds(0, CHUNK)], e_v.at[ji],
                                  sem_in.at[ji]).wait()
            pltpu.async_copy(h_hbm.at[src_v.at[ji]], h_v.at[jh], sem_g.at[jh])

    start_in(0, 0)
    start_in(1, 1)
    start_gather(0, 0, 0)

    @pl.loop(0, LOOP_HI, step=UNROLL)
    def _(i0):
        for k in range(UNROLL):
            i = i0 + k
            ji, jh = k % NBI, k % NBH
            start_in(i + 2, (k + 2) % NBI)
            start_gather(i + 1, (k + 1) % NBI, (k + 1) % NBH)

            @pl.when(w + NW * i < NCHUNKS)
            def _():
                pltpu.make_async_copy(h_hbm.at[src_v.at[ji]], h_v.at[jh],
                                      sem_g.at[jh]).wait()

                @pl.loop(0, CHUNK, step=2)
                def _(r):
                    for rr in range(2):
                        for kk in range(H // 16):
                            sl = (jh, pl.ds(r + rr, 1), pl.ds(kk * 16, 16))
                            h_v[sl] = jnp.maximum(h_v[sl] + e_v[ji, sl[1], sl[2]],
                                                  0.0)

                pltpu.sync_copy(h_v.at[jh], acc.at[dst_v.at[ji]], add=True)

    plsc.subcore_barrier()

    pltpu.sync_copy(acc.at[pl.ds(s * ROWS_PER_SUB, ROWS_PER_SUB)],
                    out_hbm.at[c, pl.ds(s * ROWS_PER_SUB, ROWS_PER_SUB)])

    @pl.when(s == 0)
    def _():
        pltpu.sync_copy(acc.at[pl.ds(NS * ROWS_PER_SUB, TAIL_ROWS)],
                        out_hbm.at[c, pl.ds(NS * ROWS_PER_SUB, TAIL_ROWS)])


@jax.jit
def _edge_pass(h, e, src, dst):
    mesh = plsc.VectorSubcoreMesh(core_axis_name="core", subcore_axis_name="subcore")
    f = pl.kernel(
        _edge_pass_body,
        out_type=jax.ShapeDtypeStruct((NC, N, H), jnp.float32),
        mesh=mesh,
        scratch_types=[
            pltpu.VMEM_SHARED((N, H), jnp.float32),
            pltpu.VMEM((NBI, CHUNK), jnp.int32),
            pltpu.VMEM((NBI, CHUNK), jnp.int32),
            pltpu.VMEM((NBH, CHUNK, H), jnp.float32),
            pltpu.VMEM((NBI, CHUNK, H), jnp.float32),
            pltpu.SemaphoreType.DMA((NBI,)),
            pltpu.SemaphoreType.DMA((NBH,)),
        ],
        name="gine_edge_pass",
    )
    return f(h, e, src, dst)


def _encode_h_body(x_ref, wn_ref, bn_ref, out_ref):
    out_ref[...] = jnp.dot(x_ref[...], wn_ref[...],
                           preferred_element_type=jnp.float32,
                           precision=lax.Precision.HIGHEST) + bn_ref[...]


def _encode_e_body(ea_ref, we_ref, be_ref, out_ref):
    out_ref[...] = jnp.dot(ea_ref[...], we_ref[...],
                           preferred_element_type=jnp.float32) + be_ref[...]


def _mlp_core(h_ref, parts_ref, w1_ref, b1_ref, w2_ref, b2_ref, g_ref, bt_ref):
    z = h_ref[...] + parts_ref[0] + parts_ref[1]
    z = jnp.maximum(jnp.dot(z, w1_ref[...], preferred_element_type=jnp.float32,
                            precision=lax.Precision.HIGHEST) + b1_ref[...], 0.0)
    z = jnp.dot(z, w2_ref[...], preferred_element_type=jnp.float32,
                precision=lax.Precision.HIGHEST) + b2_ref[...]
    mu = jnp.mean(z, axis=0, keepdims=True)
    var = jnp.mean(z * z, axis=0, keepdims=True) - mu * mu
    zn = (z - mu) * lax.rsqrt(var + 1e-5) * g_ref[...] + bt_ref[...]
    return jnp.maximum(zn, 0.0)


def _mlp_bn_body(h_ref, parts_ref, w1_ref, b1_ref, w2_ref, b2_ref,
                 g_ref, bt_ref, out_ref):
    out_ref[...] = _mlp_core(h_ref, parts_ref, w1_ref, b1_ref, w2_ref, b2_ref,
                             g_ref, bt_ref)


def _mlp_bn_pool_body(h_ref, parts_ref, w1_ref, b1_ref, w2_ref, b2_ref,
                      g_ref, bt_ref, b_ref, wf_ref, bf_ref, out_ref):
    hf = _mlp_core(h_ref, parts_ref, w1_ref, b1_ref, w2_ref, b2_ref,
                   g_ref, bt_ref)
    gids = lax.broadcasted_iota(jnp.int32, (G, N), 0)
    oh = (gids == b_ref[...]).astype(jnp.float32)
    sums = jnp.dot(oh, hf, preferred_element_type=jnp.float32,
                   precision=lax.Precision.HIGHEST)
    cnt = jnp.sum(oh, axis=1, keepdims=True)
    mean = sums / jnp.maximum(cnt, 1.0)
    out_ref[...] = jnp.dot(mean, wf_ref[...], preferred_element_type=jnp.float32,
                           precision=lax.Precision.HIGHEST) + bf_ref[...]


def _tc_call(body, out_shape, *args):
    return pl.pallas_call(body, out_shape=out_shape)(*args)


def kernel(x, edge_index, edge_attr, batch, Wn, bn_, We, be,
           l0_w1, l0_b1, l0_w2, l0_b2, l0_g, l0_bt,
           l1_w1, l1_b1, l1_w2, l1_b2, l1_g, l1_bt, Wf, bf):
    src = edge_index[0]
    dst = edge_index[1]

    h = _tc_call(_encode_h_body, jax.ShapeDtypeStruct((N, H), jnp.float32),
                 x, Wn, bn_.reshape(1, H))

    PK = 8
    W8 = jnp.zeros((PK, DE, PK, H), jnp.float32)
    W8 = W8.at[jnp.arange(PK), :, jnp.arange(PK), :].set(
        jnp.broadcast_to(We, (PK, DE, H))).reshape(PK * DE, PK * H)
    EB = 2000
    e = pl.pallas_call(
        _encode_e_body,
        out_shape=jax.ShapeDtypeStruct((E // PK, PK * H), jnp.float32),
        grid=(E // PK // EB,),
        in_specs=[pl.BlockSpec((EB, PK * DE), lambda i: (i, 0)),
                  pl.BlockSpec((PK * DE, PK * H), lambda i: (0, 0)),
                  pl.BlockSpec((1, PK * H), lambda i: (0, 0))],
        out_specs=pl.BlockSpec((EB, PK * H), lambda i: (i, 0)),
    )(edge_attr.reshape(E // PK, PK * DE).astype(jnp.bfloat16),
      W8.astype(jnp.bfloat16),
      jnp.tile(be, (PK,)).reshape(1, PK * H)).reshape(E, H)

    (w1, b1, w2, b2, g, bt) = (l0_w1, l0_b1, l0_w2, l0_b2, l0_g, l0_bt)
    parts = _edge_pass(h, e, src, dst)
    h = _tc_call(_mlp_bn_body, jax.ShapeDtypeStruct((N, H), jnp.float32),
                 h, parts, w1, b1.reshape(1, H), w2, b2.reshape(1, H),
                 g.reshape(1, H), bt.reshape(1, H))

    (w1, b1, w2, b2, g, bt) = (l1_w1, l1_b1, l1_w2, l1_b2, l1_g, l1_bt)
    parts = _edge_pass(h, e, src, dst)
    return _tc_call(_mlp_bn_pool_body, jax.ShapeDtypeStruct((G, EMB), jnp.float32),
                    h, parts, w1, b1.reshape(1, H), w2, b2.reshape(1, H),
                    g.reshape(1, H), bt.reshape(1, H),
                    batch.reshape(1, N), Wf, bf.reshape(1, EMB))


# speedup vs baseline: 1.3829x; 1.2577x over previous
import functools

import jax
import jax.numpy as jnp
from jax import lax
from jax.experimental import pallas as pl
from jax.experimental.pallas import tpu as pltpu
from jax.experimental.pallas import tpu_sc as plsc

N = 10000
E = 320000
DF = 128
DE = 16
H = 128
EMB = 128
G = 64

NC = 2
NS = 16
CHUNK = 64
NCHUNKS = E // CHUNK
CPW = -(-NCHUNKS // (NC * NS))
ROWS_PER_SUB = 624
TAIL_ROWS = N - NS * ROWS_PER_SUB


NW = NC * NS
NBI = 3
NBH = 2
UNROLL = 6
LOOP_HI = UNROLL * (-(-(CPW + 2) // UNROLL))


def _edge_pass_body(h_hbm, e_hbm, src_hbm, dst_hbm, out_hbm,
                    acc, src_v, dst_v, h_v, e_v, sem_in, sem_g):
    c = lax.axis_index("core")
    s = lax.axis_index("subcore")
    w = c * NS + s

    @pl.loop(0, CHUNK)
    def _(r):
        for k in range(H // 16):
            e_v[0, pl.ds(r, 1), pl.ds(k * 16, 16)] = jnp.zeros((1, 16), jnp.float32)

    _nfull, _rem = divmod(ROWS_PER_SUB, CHUNK)
    for j in range(_nfull):
        pltpu.sync_copy(e_v.at[0],
                        acc.at[pl.ds(s * ROWS_PER_SUB + j * CHUNK, CHUNK)])
    if _rem:
        pltpu.sync_copy(e_v.at[0, pl.ds(0, _rem)],
                        acc.at[pl.ds(s * ROWS_PER_SUB + _nfull * CHUNK, _rem)])

    @pl.when(s == 0)
    def _():
        pltpu.sync_copy(e_v.at[0, pl.ds(0, TAIL_ROWS)],
                        acc.at[pl.ds(NS * ROWS_PER_SUB, TAIL_ROWS)])

    plsc.subcore_barrier()

    def start_in(i, ji):
        @pl.when(w + NW * i < NCHUNKS)
        def _():
            base = (w + NW * i) * CHUNK
            pltpu.async_copy(src_hbm.at[pl.ds(base, CHUNK)], src_v.at[ji],
                             sem_in.at[ji])
            pltpu.async_copy(dst_hbm.at[pl.ds(base, CHUNK)], dst_v.at[ji],
                             sem_in.at[ji])
            pltpu.async_copy(e_hbm.at[pl.ds(base, CHUNK)], e_v.at[ji],
                             sem_in.at[ji])

    def start_gather(i, ji, jh):
        @pl.when(w + NW * i < NCHUNKS)
        def _():
            pltpu.make_async_copy(src_hbm.at[pl.ds(0, CHUNK)], src_v.at[ji],
                                  sem_in.at[ji]).wait()
            pltpu.make_async_copy(dst_hbm.at[pl.ds(0, CHUNK)], dst_v.at[ji],
                                  sem_in.at[ji]).wait()
            pltpu.make_async_copy(e_hbm.at[pl.ds(0, CHUNK)], e_v.at[ji],
                                  sem_in.at[ji]).wait()
            pltpu.async_copy(h_hbm.at[src_v.at[ji]], h_v.at[jh], sem_g.at[jh])

    start_in(0, 0)
    start_in(1, 1)
    start_gather(0, 0, 0)

    @pl.loop(0, LOOP_HI, step=UNROLL)
    def _(i0):
        for k in range(UNROLL):
            i = i0 + k
            ji, jh = k % NBI, k % NBH
            start_in(i + 2, (k + 2) % NBI)
            start_gather(i + 1, (k + 1) % NBI, (k + 1) % NBH)

            @pl.when(w + NW * i < NCHUNKS)
            def _():
                pltpu.make_async_copy(h_hbm.at[src_v.at[ji]], h_v.at[jh],
                                      sem_g.at[jh]).wait()

                @pl.loop(0, CHUNK, step=2)
                def _(r):
                    for rr in range(2):
                        for kk in range(H // 16):
                            sl = (jh, pl.ds(r + rr, 1), pl.ds(kk * 16, 16))
                            h_v[sl] = jnp.maximum(h_v[sl] + e_v[ji, sl[1], sl[2]],
                                                  0.0)

                pltpu.sync_copy(h_v.at[jh], acc.at[dst_v.at[ji]], add=True)

    plsc.subcore_barrier()

    pltpu.sync_copy(acc.at[pl.ds(s * ROWS_PER_SUB, ROWS_PER_SUB)],
                    out_hbm.at[c, pl.ds(s * ROWS_PER_SUB, ROWS_PER_SUB)])

    @pl.when(s == 0)
    def _():
        pltpu.sync_copy(acc.at[pl.ds(NS * ROWS_PER_SUB, TAIL_ROWS)],
                        out_hbm.at[c, pl.ds(NS * ROWS_PER_SUB, TAIL_ROWS)])


@jax.jit
def _edge_pass(h, e, src, dst):
    mesh = plsc.VectorSubcoreMesh(core_axis_name="core", subcore_axis_name="subcore")
    f = pl.kernel(
        _edge_pass_body,
        out_type=jax.ShapeDtypeStruct((NC, N, H), jnp.float32),
        mesh=mesh,
        scratch_types=[
            pltpu.VMEM_SHARED((N, H), jnp.float32),
            pltpu.VMEM((NBI, CHUNK), jnp.int32),
            pltpu.VMEM((NBI, CHUNK), jnp.int32),
            pltpu.VMEM((NBH, CHUNK, H), jnp.float32),
            pltpu.VMEM((NBI, CHUNK, H), jnp.float32),
            pltpu.SemaphoreType.DMA((NBI,)),
            pltpu.SemaphoreType.DMA((NBH,)),
        ],
        name="gine_edge_pass",
    )
    return f(h, e, src, dst)


def _encode_h_body(x_ref, wn_ref, bn_ref, out_ref):
    out_ref[...] = jnp.dot(x_ref[...], wn_ref[...],
                           preferred_element_type=jnp.float32,
                           precision=lax.Precision.HIGHEST) + bn_ref[...]


def _encode_e_body(ea_ref, we_ref, be_ref, out_ref):
    out_ref[...] = jnp.dot(ea_ref[...], we_ref[...],
                           preferred_element_type=jnp.float32) + be_ref[...]


def _mlp_core(h_ref, parts_ref, w1_ref, b1_ref, w2_ref, b2_ref, g_ref, bt_ref):
    z = h_ref[...] + parts_ref[0] + parts_ref[1]
    z = jnp.maximum(jnp.dot(z, w1_ref[...], preferred_element_type=jnp.float32,
                            precision=lax.Precision.HIGHEST) + b1_ref[...], 0.0)
    z = jnp.dot(z, w2_ref[...], preferred_element_type=jnp.float32,
                precision=lax.Precision.HIGHEST) + b2_ref[...]
    mu = jnp.mean(z, axis=0, keepdims=True)
    var = jnp.mean(z * z, axis=0, keepdims=True) - mu * mu
    zn = (z - mu) * lax.rsqrt(var + 1e-5) * g_ref[...] + bt_ref[...]
    return jnp.maximum(zn, 0.0)


def _mlp_bn_body(h_ref, parts_ref, w1_ref, b1_ref, w2_ref, b2_ref,
                 g_ref, bt_ref, out_ref):
    out_ref[...] = _mlp_core(h_ref, parts_ref, w1_ref, b1_ref, w2_ref, b2_ref,
                             g_ref, bt_ref)


def _mlp_bn_pool_body(h_ref, parts_ref, w1_ref, b1_ref, w2_ref, b2_ref,
                      g_ref, bt_ref, b_ref, wf_ref, bf_ref, out_ref):
    hf = _mlp_core(h_ref, parts_ref, w1_ref, b1_ref, w2_ref, b2_ref,
                   g_ref, bt_ref)
    gids = lax.broadcasted_iota(jnp.int32, (G, N), 0)
    oh = (gids == b_ref[...]).astype(jnp.float32)
    sums = jnp.dot(oh, hf, preferred_element_type=jnp.float32,
                   precision=lax.Precision.HIGHEST)
    cnt = jnp.sum(oh, axis=1, keepdims=True)
    mean = sums / jnp.maximum(cnt, 1.0)
    out_ref[...] = jnp.dot(mean, wf_ref[...], preferred_element_type=jnp.float32,
                           precision=lax.Precision.HIGHEST) + bf_ref[...]


def _tc_call(body, out_shape, *args):
    return pl.pallas_call(body, out_shape=out_shape)(*args)


def kernel(x, edge_index, edge_attr, batch, Wn, bn_, We, be,
           l0_w1, l0_b1, l0_w2, l0_b2, l0_g, l0_bt,
           l1_w1, l1_b1, l1_w2, l1_b2, l1_g, l1_bt, Wf, bf):
    src = edge_index[0]
    dst = edge_index[1]

    h = _tc_call(_encode_h_body, jax.ShapeDtypeStruct((N, H), jnp.float32),
                 x, Wn, bn_.reshape(1, H))

    EB = 8000
    e = pl.pallas_call(
        _encode_e_body,
        out_shape=jax.ShapeDtypeStruct((E, H), jnp.float32),
        grid=(E // EB,),
        in_specs=[pl.BlockSpec((EB, DE), lambda i: (i, 0)),
                  pl.BlockSpec((DE, H), lambda i: (0, 0)),
                  pl.BlockSpec((1, H), lambda i: (0, 0))],
        out_specs=pl.BlockSpec((EB, H), lambda i: (i, 0)),
    )(edge_attr.astype(jnp.bfloat16), We.astype(jnp.bfloat16),
      be.reshape(1, H))

    (w1, b1, w2, b2, g, bt) = (l0_w1, l0_b1, l0_w2, l0_b2, l0_g, l0_bt)
    parts = _edge_pass(h, e, src, dst)
    h = _tc_call(_mlp_bn_body, jax.ShapeDtypeStruct((N, H), jnp.float32),
                 h, parts, w1, b1.reshape(1, H), w2, b2.reshape(1, H),
                 g.reshape(1, H), bt.reshape(1, H))

    (w1, b1, w2, b2, g, bt) = (l1_w1, l1_b1, l1_w2, l1_b2, l1_g, l1_bt)
    parts = _edge_pass(h, e, src, dst)
    return _tc_call(_mlp_bn_pool_body, jax.ShapeDtypeStruct((G, EMB), jnp.float32),
                    h, parts, w1, b1.reshape(1, H), w2, b2.reshape(1, H),
                    g.reshape(1, H), bt.reshape(1, H),
                    batch.reshape(1, N), Wf, bf.reshape(1, EMB))


# speedup vs baseline: 1.4604x; 1.0560x over previous
import functools

import jax
import jax.numpy as jnp
from jax import lax
from jax.experimental import pallas as pl
from jax.experimental.pallas import tpu as pltpu
from jax.experimental.pallas import tpu_sc as plsc

N = 10000
E = 320000
DF = 128
DE = 16
H = 128
EMB = 128
G = 64

NC = 2
NS = 16
CHUNK = 64
NCHUNKS = E // CHUNK
CPW = -(-NCHUNKS // (NC * NS))
ROWS_PER_SUB = 624
TAIL_ROWS = N - NS * ROWS_PER_SUB


NW = NC * NS
NBI = 3
NBD = 4
NBH = 3
NBS = 2
UNROLL = 12
LOOP_HI = UNROLL * (-(-(CPW + 2) // UNROLL))


def _edge_pass_body(h_hbm, e_hbm, src_hbm, dst_hbm, out_hbm,
                    acc, src_v, dst_v, h_v, e_v, sem_in, sem_g, sem_s):
    c = lax.axis_index("core")
    s = lax.axis_index("subcore")
    w = c * NS + s

    @pl.loop(0, CHUNK)
    def _(r):
        for k in range(H // 16):
            e_v[0, pl.ds(r, 1), pl.ds(k * 16, 16)] = jnp.zeros((1, 16), jnp.float32)

    _nfull, _rem = divmod(ROWS_PER_SUB, CHUNK)
    for j in range(_nfull):
        pltpu.sync_copy(e_v.at[0],
                        acc.at[pl.ds(s * ROWS_PER_SUB + j * CHUNK, CHUNK)])
    if _rem:
        pltpu.sync_copy(e_v.at[0, pl.ds(0, _rem)],
                        acc.at[pl.ds(s * ROWS_PER_SUB + _nfull * CHUNK, _rem)])

    @pl.when(s == 0)
    def _():
        pltpu.sync_copy(e_v.at[0, pl.ds(0, TAIL_ROWS)],
                        acc.at[pl.ds(NS * ROWS_PER_SUB, TAIL_ROWS)])

    plsc.subcore_barrier()

    def start_in(i, ji, jd):
        @pl.when(w + NW * i < NCHUNKS)
        def _():
            base = (w + NW * i) * CHUNK
            pltpu.async_copy(src_hbm.at[pl.ds(base, CHUNK)], src_v.at[ji],
                             sem_in.at[ji])
            pltpu.async_copy(dst_hbm.at[pl.ds(base, CHUNK)], dst_v.at[jd],
                             sem_in.at[ji])
            pltpu.async_copy(e_hbm.at[pl.ds(base, CHUNK)], e_v.at[ji],
                             sem_in.at[ji])

    def start_gather(i, ji, jd, jh):
        @pl.when(w + NW * i < NCHUNKS)
        def _():
            pltpu.make_async_copy(src_hbm.at[pl.ds(0, CHUNK)], src_v.at[ji],
                                  sem_in.at[ji]).wait()
            pltpu.make_async_copy(dst_hbm.at[pl.ds(0, CHUNK)], dst_v.at[jd],
                                  sem_in.at[ji]).wait()
            pltpu.make_async_copy(e_hbm.at[pl.ds(0, CHUNK)], e_v.at[ji],
                                  sem_in.at[ji]).wait()
            pltpu.async_copy(h_hbm.at[src_v.at[ji]], h_v.at[jh], sem_g.at[jh])

    def wait_scatter(i, jh, jd, js):
        @pl.when((i >= 0) & (w + NW * i < NCHUNKS))
        def _():
            pltpu.make_async_copy(h_v.at[jh], acc.at[dst_v.at[jd]],
                                  sem_s.at[js]).wait()

    start_in(0, 0, 0)
    start_in(1, 1, 1)
    start_gather(0, 0, 0, 0)

    @pl.loop(0, LOOP_HI, step=UNROLL)
    def _(i0):
        for k in range(UNROLL):
            i = i0 + k
            ji, jd, jh, js = k % NBI, k % NBD, k % NBH, k % NBS
            wait_scatter(i - 2, (k - 2) % NBH, (k - 2) % NBD, (k - 2) % NBS)
            start_in(i + 2, (k + 2) % NBI, (k + 2) % NBD)
            start_gather(i + 1, (k + 1) % NBI, (k + 1) % NBD, (k + 1) % NBH)

            @pl.when(w + NW * i < NCHUNKS)
            def _():
                pltpu.make_async_copy(h_hbm.at[src_v.at[ji]], h_v.at[jh],
                                      sem_g.at[jh]).wait()

                @pl.loop(0, CHUNK, step=2)
                def _(r):
                    for rr in range(2):
                        for kk in range(H // 16):
                            sl = (jh, pl.ds(r + rr, 1), pl.ds(kk * 16, 16))
                            h_v[sl] = jnp.maximum(h_v[sl] + e_v[ji, sl[1], sl[2]],
                                                  0.0)

                pltpu.async_copy(h_v.at[jh], acc.at[dst_v.at[jd]], sem_s.at[js],
                                 add=True)

    plsc.subcore_barrier()

    pltpu.sync_copy(acc.at[pl.ds(s * ROWS_PER_SUB, ROWS_PER_SUB)],
                    out_hbm.at[c, pl.ds(s * ROWS_PER_SUB, ROWS_PER_SUB)])

    @pl.when(s == 0)
    def _():
        pltpu.sync_copy(acc.at[pl.ds(NS * ROWS_PER_SUB, TAIL_ROWS)],
                        out_hbm.at[c, pl.ds(NS * ROWS_PER_SUB, TAIL_ROWS)])


@jax.jit
def _edge_pass(h, e, src, dst):
    mesh = plsc.VectorSubcoreMesh(core_axis_name="core", subcore_axis_name="subcore")
    f = pl.kernel(
        _edge_pass_body,
        out_type=jax.ShapeDtypeStruct((NC, N, H), jnp.float32),
        mesh=mesh,
        scratch_types=[
            pltpu.VMEM_SHARED((N, H), jnp.float32),
            pltpu.VMEM((NBI, CHUNK), jnp.int32),
            pltpu.VMEM((NBD, CHUNK), jnp.int32),
            pltpu.VMEM((NBH, CHUNK, H), jnp.float32),
            pltpu.VMEM((NBI, CHUNK, H), jnp.float32),
            pltpu.SemaphoreType.DMA((NBI,)),
            pltpu.SemaphoreType.DMA((NBH,)),
            pltpu.SemaphoreType.DMA((NBS,)),
        ],
        name="gine_edge_pass",
    )
    return f(h, e, src, dst)


def _encode_h_body(x_ref, wn_ref, bn_ref, out_ref):
    out_ref[...] = jnp.dot(x_ref[...], wn_ref[...],
                           preferred_element_type=jnp.float32,
                           precision=lax.Precision.HIGHEST) + bn_ref[...]


def _encode_e_body(ea_ref, we_ref, be_ref, out_ref):
    out_ref[...] = jnp.dot(ea_ref[...], we_ref[...],
                           preferred_element_type=jnp.float32) + be_ref[...]


def _mlp_core(h_ref, parts_ref, w1_ref, b1_ref, w2_ref, b2_ref, g_ref, bt_ref):
    z = h_ref[...] + parts_ref[0] + parts_ref[1]
    z = jnp.maximum(jnp.dot(z, w1_ref[...], preferred_element_type=jnp.float32,
                            precision=lax.Precision.HIGHEST) + b1_ref[...], 0.0)
    z = jnp.dot(z, w2_ref[...], preferred_element_type=jnp.float32,
                precision=lax.Precision.HIGHEST) + b2_ref[...]
    mu = jnp.mean(z, axis=0, keepdims=True)
    var = jnp.mean(z * z, axis=0, keepdims=True) - mu * mu
    zn = (z - mu) * lax.rsqrt(var + 1e-5) * g_ref[...] + bt_ref[...]
    return jnp.maximum(zn, 0.0)


def _mlp_bn_body(h_ref, parts_ref, w1_ref, b1_ref, w2_ref, b2_ref,
                 g_ref, bt_ref, out_ref):
    out_ref[...] = _mlp_core(h_ref, parts_ref, w1_ref, b1_ref, w2_ref, b2_ref,
                             g_ref, bt_ref)


def _mlp_bn_pool_body(h_ref, parts_ref, w1_ref, b1_ref, w2_ref, b2_ref,
                      g_ref, bt_ref, b_ref, wf_ref, bf_ref, out_ref):
    hf = _mlp_core(h_ref, parts_ref, w1_ref, b1_ref, w2_ref, b2_ref,
                   g_ref, bt_ref)
    gids = lax.broadcasted_iota(jnp.int32, (G, N), 0)
    oh = (gids == b_ref[...]).astype(jnp.float32)
    sums = jnp.dot(oh, hf, preferred_element_type=jnp.float32,
                   precision=lax.Precision.HIGHEST)
    cnt = jnp.sum(oh, axis=1, keepdims=True)
    mean = sums / jnp.maximum(cnt, 1.0)
    out_ref[...] = jnp.dot(mean, wf_ref[...], preferred_element_type=jnp.float32,
                           precision=lax.Precision.HIGHEST) + bf_ref[...]


def _tc_call(body, out_shape, *args):
    return pl.pallas_call(body, out_shape=out_shape)(*args)


def kernel(x, edge_index, edge_attr, batch, Wn, bn_, We, be,
           l0_w1, l0_b1, l0_w2, l0_b2, l0_g, l0_bt,
           l1_w1, l1_b1, l1_w2, l1_b2, l1_g, l1_bt, Wf, bf):
    src = edge_index[0]
    dst = edge_index[1]

    h = _tc_call(_encode_h_body, jax.ShapeDtypeStruct((N, H), jnp.float32),
                 x, Wn, bn_.reshape(1, H))

    EB = 8000
    e = pl.pallas_call(
        _encode_e_body,
        out_shape=jax.ShapeDtypeStruct((E, H), jnp.float32),
        grid=(E // EB,),
        in_specs=[pl.BlockSpec((EB, DE), lambda i: (i, 0)),
                  pl.BlockSpec((DE, H), lambda i: (0, 0)),
                  pl.BlockSpec((1, H), lambda i: (0, 0))],
        out_specs=pl.BlockSpec((EB, H), lambda i: (i, 0)),
    )(edge_attr.astype(jnp.bfloat16), We.astype(jnp.bfloat16),
      be.reshape(1, H))

    (w1, b1, w2, b2, g, bt) = (l0_w1, l0_b1, l0_w2, l0_b2, l0_g, l0_bt)
    parts = _edge_pass(h, e, src, dst)
    h = _tc_call(_mlp_bn_body, jax.ShapeDtypeStruct((N, H), jnp.float32),
                 h, parts, w1, b1.reshape(1, H), w2, b2.reshape(1, H),
                 g.reshape(1, H), bt.reshape(1, H))

    (w1, b1, w2, b2, g, bt) = (l1_w1, l1_b1, l1_w2, l1_b2, l1_g, l1_bt)
    parts = _edge_pass(h, e, src, dst)
    return _tc_call(_mlp_bn_pool_body, jax.ShapeDtypeStruct((G, EMB), jnp.float32),
                    h, parts, w1, b1.reshape(1, H), w2, b2.reshape(1, H),
                    g.reshape(1, H), bt.reshape(1, H),
                    batch.reshape(1, N), Wf, bf.reshape(1, EMB))


# speedup vs baseline: 1.4619x; 1.0010x over previous
import jax
import jax.numpy as jnp
from jax import lax
from jax.experimental import pallas as pl
from jax.experimental.pallas import tpu as pltpu
from jax.experimental.pallas import tpu_sc as plsc

N = 10000
E = 320000
DF = 128
DE = 16
H = 128
EMB = 128
G = 64

NC = 2
NS = 16
CHUNK = 64
NCHUNKS = E // CHUNK
CPW = -(-NCHUNKS // (NC * NS))
ROWS_PER_SUB = 624
TAIL_ROWS = N - NS * ROWS_PER_SUB


NW = NC * NS
NBI = 3
NBD = 4
NBH = 3
NBS = 2
UNROLL = 12
LOOP_HI = UNROLL * (-(-(CPW + 2) // UNROLL))


def _edge_pass_body(h_hbm, e_hbm, src_hbm, dst_hbm, out_hbm,
                    acc, src_v, dst_v, h_v, e_v, sem_in, sem_g, sem_s):
    c = lax.axis_index("core")
    s = lax.axis_index("subcore")
    w = c * NS + s

    @pl.loop(0, CHUNK)
    def _(r):
        for k in range(H // 16):
            e_v[0, pl.ds(r, 1), pl.ds(k * 16, 16)] = jnp.zeros((1, 16), jnp.float32)

    _nfull, _rem = divmod(ROWS_PER_SUB, CHUNK)
    for j in range(_nfull):
        pltpu.sync_copy(e_v.at[0],
                        acc.at[pl.ds(s * ROWS_PER_SUB + j * CHUNK, CHUNK)])
    if _rem:
        pltpu.sync_copy(e_v.at[0, pl.ds(0, _rem)],
                        acc.at[pl.ds(s * ROWS_PER_SUB + _nfull * CHUNK, _rem)])

    @pl.when(s == 0)
    def _():
        pltpu.sync_copy(e_v.at[0, pl.ds(0, TAIL_ROWS)],
                        acc.at[pl.ds(NS * ROWS_PER_SUB, TAIL_ROWS)])

    plsc.subcore_barrier()

    def start_in(i, ji, jd):
        @pl.when(w + NW * i < NCHUNKS)
        def _():
            base = (w + NW * i) * CHUNK
            pltpu.async_copy(src_hbm.at[pl.ds(base, CHUNK)], src_v.at[ji],
                             sem_in.at[ji])
            pltpu.async_copy(dst_hbm.at[pl.ds(base, CHUNK)], dst_v.at[jd],
                             sem_in.at[ji])
            pltpu.async_copy(e_hbm.at[pl.ds(base, CHUNK)], e_v.at[ji],
                             sem_in.at[ji])

    def start_gather(i, ji, jd, jh):
        @pl.when(w + NW * i < NCHUNKS)
        def _():
            pltpu.make_async_copy(src_hbm.at[pl.ds(0, CHUNK)], src_v.at[ji],
                                  sem_in.at[ji]).wait()
            pltpu.make_async_copy(dst_hbm.at[pl.ds(0, CHUNK)], dst_v.at[jd],
                                  sem_in.at[ji]).wait()
            pltpu.make_async_copy(e_hbm.at[pl.ds(0, CHUNK)], e_v.at[ji],
                                  sem_in.at[ji]).wait()
            pltpu.async_copy(h_hbm.at[src_v.at[ji]], h_v.at[jh], sem_g.at[jh])

    def wait_scatter(i, jh, jd, js):
        @pl.when((i >= 0) & (w + NW * i < NCHUNKS))
        def _():
            pltpu.make_async_copy(h_v.at[jh], acc.at[dst_v.at[jd]],
                                  sem_s.at[js]).wait()

    start_in(0, 0, 0)
    start_in(1, 1, 1)
    start_gather(0, 0, 0, 0)

    @pl.loop(0, LOOP_HI, step=UNROLL)
    def _(i0):
        for k in range(UNROLL):
            i = i0 + k
            ji, jd, jh, js = k % NBI, k % NBD, k % NBH, k % NBS
            wait_scatter(i - 2, (k - 2) % NBH, (k - 2) % NBD, (k - 2) % NBS)
            start_in(i + 2, (k + 2) % NBI, (k + 2) % NBD)
            start_gather(i + 1, (k + 1) % NBI, (k + 1) % NBD, (k + 1) % NBH)

            @pl.when(w + NW * i < NCHUNKS)
            def _():
                pltpu.make_async_copy(h_hbm.at[src_v.at[ji]], h_v.at[jh],
                                      sem_g.at[jh]).wait()

                @pl.loop(0, CHUNK, step=2)
                def _(r):
                    for rr in range(2):
                        for kk in range(H // 16):
                            sl = (jh, pl.ds(r + rr, 1), pl.ds(kk * 16, 16))
                            h_v[sl] = jnp.maximum(h_v[sl] + e_v[ji, sl[1], sl[2]],
                                                  0.0)

                pltpu.async_copy(h_v.at[jh], acc.at[dst_v.at[jd]], sem_s.at[js],
                                 add=True)

    plsc.subcore_barrier()

    pltpu.sync_copy(acc.at[pl.ds(s * ROWS_PER_SUB, ROWS_PER_SUB)],
                    out_hbm.at[c, pl.ds(s * ROWS_PER_SUB, ROWS_PER_SUB)])

    @pl.when(s == 0)
    def _():
        pltpu.sync_copy(acc.at[pl.ds(NS * ROWS_PER_SUB, TAIL_ROWS)],
                        out_hbm.at[c, pl.ds(NS * ROWS_PER_SUB, TAIL_ROWS)])


@jax.jit
def _edge_pass(h, e, src, dst):
    mesh = plsc.VectorSubcoreMesh(core_axis_name="core", subcore_axis_name="subcore")
    f = pl.kernel(
        _edge_pass_body,
        out_type=jax.ShapeDtypeStruct((NC, N, H), jnp.float32),
        mesh=mesh,
        scratch_types=[
            pltpu.VMEM_SHARED((N, H), jnp.float32),
            pltpu.VMEM((NBI, CHUNK), jnp.int32),
            pltpu.VMEM((NBD, CHUNK), jnp.int32),
            pltpu.VMEM((NBH, CHUNK, H), jnp.float32),
            pltpu.VMEM((NBI, CHUNK, H), jnp.float32),
            pltpu.SemaphoreType.DMA((NBI,)),
            pltpu.SemaphoreType.DMA((NBH,)),
            pltpu.SemaphoreType.DMA((NBS,)),
        ],
        name="gine_edge_pass",
    )
    return f(h, e, src, dst)


def _encode_h_body(x_ref, wn_ref, bn_ref, out_ref):
    out_ref[...] = jnp.dot(x_ref[...], wn_ref[...],
                           preferred_element_type=jnp.float32,
                           precision=lax.Precision.HIGHEST) + bn_ref[...]


def _encode_e_body(ea_ref, we_ref, be_ref, out_ref):
    out_ref[...] = jnp.dot(ea_ref[...], we_ref[...],
                           preferred_element_type=jnp.float32) + be_ref[...]


def _mlp_core(h_ref, parts_ref, w1_ref, b1_ref, w2_ref, b2_ref, g_ref, bt_ref):
    z = h_ref[...] + parts_ref[0] + parts_ref[1]
    z = jnp.maximum(jnp.dot(z, w1_ref[...], preferred_element_type=jnp.float32,
                            precision=lax.Precision.HIGHEST) + b1_ref[...], 0.0)
    z = jnp.dot(z, w2_ref[...], preferred_element_type=jnp.float32,
                precision=lax.Precision.HIGHEST) + b2_ref[...]
    mu = jnp.mean(z, axis=0, keepdims=True)
    var = jnp.mean(z * z, axis=0, keepdims=True) - mu * mu
    zn = (z - mu) * lax.rsqrt(var + 1e-5) * g_ref[...] + bt_ref[...]
    return jnp.maximum(zn, 0.0)


def _mlp_bn_body(h_ref, parts_ref, w1_ref, b1_ref, w2_ref, b2_ref,
                 g_ref, bt_ref, out_ref):
    out_ref[...] = _mlp_core(h_ref, parts_ref, w1_ref, b1_ref, w2_ref, b2_ref,
                             g_ref, bt_ref)


def _mlp_bn_pool_body(h_ref, parts_ref, w1_ref, b1_ref, w2_ref, b2_ref,
                      g_ref, bt_ref, b_ref, wf_ref, bf_ref, out_ref):
    hf = _mlp_core(h_ref, parts_ref, w1_ref, b1_ref, w2_ref, b2_ref,
                   g_ref, bt_ref)
    gids = lax.broadcasted_iota(jnp.int32, (G, N), 0)
    oh = (gids == b_ref[...]).astype(jnp.float32)
    sums = jnp.dot(oh, hf, preferred_element_type=jnp.float32,
                   precision=lax.Precision.HIGHEST)
    cnt = jnp.sum(oh, axis=1, keepdims=True)
    mean = sums / jnp.maximum(cnt, 1.0)
    out_ref[...] = jnp.dot(mean, wf_ref[...], preferred_element_type=jnp.float32,
                           precision=lax.Precision.HIGHEST) + bf_ref[...]


def _tc_call(body, out_shape, *args):
    return pl.pallas_call(body, out_shape=out_shape)(*args)


def kernel(x, edge_index, edge_attr, batch, Wn, bn_, We, be,
           l0_w1, l0_b1, l0_w2, l0_b2, l0_g, l0_bt,
           l1_w1, l1_b1, l1_w2, l1_b2, l1_g, l1_bt, Wf, bf):
    src = edge_index[0]
    dst = edge_index[1]

    h = _tc_call(_encode_h_body, jax.ShapeDtypeStruct((N, H), jnp.float32),
                 x, Wn, bn_.reshape(1, H))

    EB = 8000
    e = pl.pallas_call(
        _encode_e_body,
        out_shape=jax.ShapeDtypeStruct((E, H), jnp.float32),
        grid=(E // EB,),
        in_specs=[pl.BlockSpec((EB, DE), lambda i: (i, 0)),
                  pl.BlockSpec((DE, H), lambda i: (0, 0)),
                  pl.BlockSpec((1, H), lambda i: (0, 0))],
        out_specs=pl.BlockSpec((EB, H), lambda i: (i, 0)),
    )(edge_attr.astype(jnp.bfloat16), We.astype(jnp.bfloat16),
      be.reshape(1, H))

    (w1, b1, w2, b2, g, bt) = (l0_w1, l0_b1, l0_w2, l0_b2, l0_g, l0_bt)
    parts = _edge_pass(h, e, src, dst)
    h = _tc_call(_mlp_bn_body, jax.ShapeDtypeStruct((N, H), jnp.float32),
                 h, parts, w1, b1.reshape(1, H), w2, b2.reshape(1, H),
                 g.reshape(1, H), bt.reshape(1, H))

    (w1, b1, w2, b2, g, bt) = (l1_w1, l1_b1, l1_w2, l1_b2, l1_g, l1_bt)
    parts = _edge_pass(h, e, src, dst)
    return _tc_call(_mlp_bn_pool_body, jax.ShapeDtypeStruct((G, EMB), jnp.float32),
                    h, parts, w1, b1.reshape(1, H), w2, b2.reshape(1, H),
                    g.reshape(1, H), bt.reshape(1, H),
                    batch.reshape(1, N), Wf, bf.reshape(1, EMB))
